# baseline (speedup 1.0000x reference)
import jax
import jax.numpy as jnp
from jax import lax
from jax.experimental import pallas as pl
from jax.experimental.pallas import tpu as pltpu
from jax.experimental.pallas import tpu_sc as plsc

B = 4096
D = 64
LEN_VOCAB = 51
OUT_W = 3 * D + LEN_VOCAB

_info = plsc.get_sparse_core_info()
NC = _info.num_cores
NS = _info.num_subcores
NW = NC * NS
BPW = B // NW


def _sc_body(adv_id, brd_id, ind_id, len_id, adv_t, brd_t, ind_t, eye_t,
             out, ia, ib, ii, il, ra, rb, ri, rl, out_v, sa, sb, si, sl):
    wid = lax.axis_index("s") * NC + lax.axis_index("c")
    base = wid * BPW

    pltpu.sync_copy(adv_id.at[pl.ds(base, BPW)], ia)
    pltpu.sync_copy(brd_id.at[pl.ds(base, BPW)], ib)
    pltpu.sync_copy(ind_id.at[pl.ds(base, BPW)], ii)
    pltpu.sync_copy(len_id.at[pl.ds(base, BPW)], il)

    ca = pltpu.async_copy(adv_t.at[ia], ra, sa)
    cb = pltpu.async_copy(brd_t.at[ib], rb, sb)
    ci = pltpu.async_copy(ind_t.at[ii], ri, si)
    cl = pltpu.async_copy(eye_t.at[il], rl, sl)
    ca.wait()
    cb.wait()
    ci.wait()
    cl.wait()

    def body(r, carry):
        for c in range(4):
            out_v[r, pl.ds(c * 16, 16)] = ra[r, pl.ds(c * 16, 16)]
        for c in range(4):
            out_v[r, pl.ds(D + c * 16, 16)] = rb[r, pl.ds(c * 16, 16)]
        for c in range(4):
            out_v[r, pl.ds(2 * D + c * 16, 16)] = rl[r, pl.ds(c * 16, 16)]
        for c in range(4):
            out_v[r, pl.ds(2 * D + LEN_VOCAB + c * 16, 16)] = ri[r, pl.ds(c * 16, 16)]
        return carry

    lax.fori_loop(0, BPW, body, 0)
    pltpu.sync_copy(out_v, out.at[pl.ds(base, BPW)])


def kernel(advertiser_id, brand_id, industry, campaign_length,
           adv_table, brand_table, ind_table):
    eye = jnp.eye(LEN_VOCAB, D, dtype=jnp.float32)
    mesh = plsc.VectorSubcoreMesh(core_axis_name="c", subcore_axis_name="s")
    f = pl.kernel(
        _sc_body,
        mesh=mesh,
        compiler_params=pltpu.CompilerParams(use_tc_tiling_on_sc=False),
        out_type=jax.ShapeDtypeStruct((B, OUT_W), jnp.float32),
        scratch_types=[
            pltpu.VMEM((BPW,), jnp.int32),
            pltpu.VMEM((BPW,), jnp.int32),
            pltpu.VMEM((BPW,), jnp.int32),
            pltpu.VMEM((BPW,), jnp.int32),
            pltpu.VMEM((BPW, D), jnp.float32),
            pltpu.VMEM((BPW, D), jnp.float32),
            pltpu.VMEM((BPW, D), jnp.float32),
            pltpu.VMEM((BPW, D), jnp.float32),
            pltpu.VMEM((BPW, OUT_W), jnp.float32),
            pltpu.SemaphoreType.DMA,
            pltpu.SemaphoreType.DMA,
            pltpu.SemaphoreType.DMA,
            pltpu.SemaphoreType.DMA,
        ],
    )
    return f(advertiser_id, brand_id, industry, campaign_length,
             adv_table, brand_table, ind_table, eye)
